# initial kernel scaffold (unmeasured)
import jax
import jax.numpy as jnp
from jax import lax
from jax.experimental import pallas as pl
from jax.experimental.pallas import tpu as pltpu

N_DEV = 32
B = 2
SQ_LOC = 128
SKV = 128
H_PER = 4
DH = 64
D_MODEL = 512
CHUNK = H_PER * DH


def _gather_weights(wq_shard, wo_shard):

    def body(wq_ref, wo_ref, wq_out, wo_out, q_send, q_recv, o_send, o_recv):
        me = lax.axis_index("i")
        left = (me + N_DEV - 1) % N_DEV
        right = (me + 1) % N_DEV

        barrier = pltpu.get_barrier_semaphore()
        for nbr in (left, right):
            pl.semaphore_signal(
                barrier, inc=1, device_id=(nbr,),
                device_id_type=pl.DeviceIdType.MESH,
            )
        pl.semaphore_wait(barrier, 2)

        wq_out[me] = wq_ref[...].astype(jnp.bfloat16)
        wo_out[me] = wo_ref[...].astype(jnp.bfloat16)

        for h in range(N_DEV - 1):
            src = (me + (N_DEV - h) % N_DEV) % N_DEV
            rq = pltpu.make_async_remote_copy(
                src_ref=wq_out.at[src],
                dst_ref=wq_out.at[src],
                send_sem=q_send.at[h],
                recv_sem=q_recv.at[h],
                device_id=(right,),
                device_id_type=pl.DeviceIdType.MESH,
            )
            ro = pltpu.make_async_remote_copy(
                src_ref=wo_out.at[src],
                dst_ref=wo_out.at[src],
                send_sem=o_send.at[h],
                recv_sem=o_recv.at[h],
                device_id=(right,),
                device_id_type=pl.DeviceIdType.MESH,
            )
            rq.start()
            ro.start()
            rq.wait()
            ro.wait()

    return pl.pallas_call(
        body,
        out_shape=(
            jax.ShapeDtypeStruct((N_DEV, D_MODEL, CHUNK), jnp.bfloat16),
            jax.ShapeDtypeStruct((N_DEV, CHUNK, D_MODEL), jnp.bfloat16),
        ),
        in_specs=[
            pl.BlockSpec(memory_space=pltpu.VMEM),
            pl.BlockSpec(memory_space=pltpu.VMEM),
        ],
        out_specs=(
            pl.BlockSpec(memory_space=pltpu.VMEM),
            pl.BlockSpec(memory_space=pltpu.VMEM),
        ),
        scratch_shapes=[
            pltpu.SemaphoreType.DMA((N_DEV - 1,)),
            pltpu.SemaphoreType.DMA((N_DEV - 1,)),
            pltpu.SemaphoreType.DMA((N_DEV - 1,)),
            pltpu.SemaphoreType.DMA((N_DEV - 1,)),
        ],
        compiler_params=pltpu.CompilerParams(collective_id=0),
    )(wq_shard, wo_shard)


def _attention(x, wq_all, wo_all, k_ext, v_ext):

    def body(x_ref, wq_ref, wo_ref, k_ref, v_ref, out_ref):
        c = pl.program_id(0)
        me = lax.axis_index("i")

        x2 = x_ref[...].reshape(B * SQ_LOC, D_MODEL).astype(jnp.bfloat16)
        q = jnp.dot(x2, wq_ref[0], preferred_element_type=jnp.float32)

        r = lax.broadcasted_iota(jnp.int32, (SQ_LOC, SKV), 0)
        jj = lax.broadcasted_iota(jnp.int32, (SQ_LOC, SKV), 1)
        qb = me * 2 + r // 64
        kb = jj // 64
        mask = (qb == kb) | ((qb % 4) == (kb % 4))
        row_keep = jnp.max(mask.astype(jnp.float32), axis=1, keepdims=True) > 0.5

        ctx_parts = []
        for b in range(B):
            head_parts = []
            for h in range(H_PER):
                q_bh = q[b * SQ_LOC:(b + 1) * SQ_LOC, h * DH:(h + 1) * DH]
                k_bh = k_ref[b, :, h, :].astype(jnp.bfloat16)
                v_bh = v_ref[b, :, h, :].astype(jnp.bfloat16)
                scores = lax.dot_general(
                    q_bh.astype(jnp.bfloat16), k_bh,
                    (((1,), (1,)), ((), ())),
                    preferred_element_type=jnp.float32,
                ) * 0.125
                scores = jnp.where(mask, scores, -1e9)
                m = jnp.max(scores, axis=1, keepdims=True)
                w = jnp.exp(scores - m)
                s = jnp.sum(w, axis=1, keepdims=True)
                s = jnp.where(row_keep, s, 1.0)
                w = w / s
                w = jnp.where(row_keep, w, 0.0)
                ctx_bh = jnp.dot(
                    w.astype(jnp.bfloat16), v_bh,
                    preferred_element_type=jnp.float32,
                )
                head_parts.append(ctx_bh)
            ctx_parts.append(jnp.concatenate(head_parts, axis=1))
        ctx = jnp.concatenate(ctx_parts, axis=0).astype(jnp.bfloat16)

        contrib = jnp.dot(
            ctx, wo_ref[0], preferred_element_type=jnp.float32
        )

        @pl.when(c == 0)
        def _():
            out_ref[...] = jnp.zeros_like(out_ref)

        out_ref[...] += contrib.reshape(B, SQ_LOC, D_MODEL)

    return pl.pallas_call(
        body,
        grid=(N_DEV,),
        out_shape=jax.ShapeDtypeStruct((B, SQ_LOC, D_MODEL), jnp.float32),
        in_specs=[
            pl.BlockSpec((B, SQ_LOC, D_MODEL), lambda c: (0, 0, 0)),
            pl.BlockSpec((1, D_MODEL, CHUNK), lambda c: (c, 0, 0)),
            pl.BlockSpec((1, CHUNK, D_MODEL), lambda c: (c, 0, 0)),
            pl.BlockSpec((B, SKV, H_PER, DH), lambda c: (0, 0, c, 0)),
            pl.BlockSpec((B, SKV, H_PER, DH), lambda c: (0, 0, c, 0)),
        ],
        out_specs=pl.BlockSpec((B, SQ_LOC, D_MODEL), lambda c: (0, 0, 0)),
        compiler_params=pltpu.CompilerParams(
            dimension_semantics=("arbitrary",),
        ),
    )(x, wq_all, wo_all, k_ext, v_ext)


def kernel(x, Wq, K_ext, V_ext, Wo):
    wq_all, wo_all = _gather_weights(Wq, Wo)
    return _attention(x, wq_all, wo_all, K_ext, V_ext)


# baseline (device time: 355585 ns/iter reference)
import jax
import jax.numpy as jnp
from jax import lax
from jax.experimental import pallas as pl
from jax.experimental.pallas import tpu as pltpu

N_DEV = 32
B = 2
SQ_LOC = 128
SKV = 128
H_PER = 4
DH = 64
D_MODEL = 512
CHUNK = H_PER * DH


def _gather_weights(wq_shard, wo_shard):

    def body(wq_ref, wo_ref, wq_out, wo_out, q_send, q_recv, o_send, o_recv):
        me = lax.axis_index("i")
        left = (me + N_DEV - 1) % N_DEV
        right = (me + 1) % N_DEV

        barrier = pltpu.get_barrier_semaphore()
        for nbr in (left, right):
            pl.semaphore_signal(
                barrier, inc=1, device_id=(nbr,),
                device_id_type=pl.DeviceIdType.MESH,
            )
        pl.semaphore_wait(barrier, 2)

        wq_out[me] = wq_ref[...].astype(jnp.bfloat16)
        wo_out[me] = wo_ref[...].astype(jnp.bfloat16)

        for h in range(N_DEV - 1):
            src = (me + (N_DEV - h) % N_DEV) % N_DEV
            rq = pltpu.make_async_remote_copy(
                src_ref=wq_out.at[src],
                dst_ref=wq_out.at[src],
                send_sem=q_send.at[h],
                recv_sem=q_recv.at[h],
                device_id=(right,),
                device_id_type=pl.DeviceIdType.MESH,
            )
            ro = pltpu.make_async_remote_copy(
                src_ref=wo_out.at[src],
                dst_ref=wo_out.at[src],
                send_sem=o_send.at[h],
                recv_sem=o_recv.at[h],
                device_id=(right,),
                device_id_type=pl.DeviceIdType.MESH,
            )
            rq.start()
            ro.start()
            rq.wait()
            ro.wait()

    return pl.pallas_call(
        body,
        out_shape=(
            jax.ShapeDtypeStruct((N_DEV, D_MODEL, CHUNK), jnp.bfloat16),
            jax.ShapeDtypeStruct((N_DEV, CHUNK, D_MODEL), jnp.bfloat16),
        ),
        in_specs=[
            pl.BlockSpec(memory_space=pltpu.VMEM),
            pl.BlockSpec(memory_space=pltpu.VMEM),
        ],
        out_specs=(
            pl.BlockSpec(memory_space=pltpu.VMEM),
            pl.BlockSpec(memory_space=pltpu.VMEM),
        ),
        scratch_shapes=[
            pltpu.SemaphoreType.DMA((N_DEV - 1,)),
            pltpu.SemaphoreType.DMA((N_DEV - 1,)),
            pltpu.SemaphoreType.DMA((N_DEV - 1,)),
            pltpu.SemaphoreType.DMA((N_DEV - 1,)),
        ],
        compiler_params=pltpu.CompilerParams(collective_id=0),
    )(wq_shard, wo_shard)


def _attention(x, wq_all, wo_all, k_ext, v_ext):

    def body(x_ref, wq_ref, wo_ref, k_ref, v_ref, out_ref):
        c = pl.program_id(0)
        me = lax.axis_index("i")

        x2 = x_ref[...].reshape(B * SQ_LOC, D_MODEL).astype(jnp.bfloat16)
        q = jnp.dot(x2, wq_ref[0], preferred_element_type=jnp.float32)

        r = lax.broadcasted_iota(jnp.int32, (SQ_LOC, SKV), 0)
        jj = lax.broadcasted_iota(jnp.int32, (SQ_LOC, SKV), 1)
        qb = me * 2 + r // 64
        kb = jj // 64
        mask = (qb == kb) | ((qb % 4) == (kb % 4))
        row_keep = jnp.max(mask.astype(jnp.float32), axis=1, keepdims=True) > 0.5

        ctx_parts = []
        for b in range(B):
            head_parts = []
            for h in range(H_PER):
                q_bh = q[b * SQ_LOC:(b + 1) * SQ_LOC, h * DH:(h + 1) * DH]
                k_bh = k_ref[b, h, :, :].astype(jnp.bfloat16)
                v_bh = v_ref[b, h, :, :].astype(jnp.bfloat16)
                scores = lax.dot_general(
                    q_bh.astype(jnp.bfloat16), k_bh,
                    (((1,), (1,)), ((), ())),
                    preferred_element_type=jnp.float32,
                ) * 0.125
                scores = jnp.where(mask, scores, -1e9)
                m = jnp.max(scores, axis=1, keepdims=True)
                w = jnp.exp(scores - m)
                s = jnp.sum(w, axis=1, keepdims=True)
                s = jnp.where(row_keep, s, 1.0)
                w = w / s
                w = jnp.where(row_keep, w, 0.0)
                ctx_bh = jnp.dot(
                    w.astype(jnp.bfloat16), v_bh,
                    preferred_element_type=jnp.float32,
                )
                head_parts.append(ctx_bh)
            ctx_parts.append(jnp.concatenate(head_parts, axis=1))
        ctx = jnp.concatenate(ctx_parts, axis=0).astype(jnp.bfloat16)

        contrib = jnp.dot(
            ctx, wo_ref[0], preferred_element_type=jnp.float32
        )

        @pl.when(c == 0)
        def _():
            out_ref[...] = jnp.zeros_like(out_ref)

        out_ref[...] += contrib.reshape(B, SQ_LOC, D_MODEL)

    return pl.pallas_call(
        body,
        grid=(N_DEV,),
        out_shape=jax.ShapeDtypeStruct((B, SQ_LOC, D_MODEL), jnp.float32),
        in_specs=[
            pl.BlockSpec((B, SQ_LOC, D_MODEL), lambda c: (0, 0, 0)),
            pl.BlockSpec((1, D_MODEL, CHUNK), lambda c: (c, 0, 0)),
            pl.BlockSpec((1, CHUNK, D_MODEL), lambda c: (c, 0, 0)),
            pl.BlockSpec((B, H_PER, SKV, DH), lambda c: (0, c, 0, 0)),
            pl.BlockSpec((B, H_PER, SKV, DH), lambda c: (0, c, 0, 0)),
        ],
        out_specs=pl.BlockSpec((B, SQ_LOC, D_MODEL), lambda c: (0, 0, 0)),
        compiler_params=pltpu.CompilerParams(
            dimension_semantics=("arbitrary",),
        ),
    )(x, wq_all, wo_all, k_ext, v_ext)


def kernel(x, Wq, K_ext, V_ext, Wo):
    wq_all, wo_all = _gather_weights(Wq, Wo)
    k_hm = jnp.transpose(K_ext, (0, 2, 1, 3))
    v_hm = jnp.transpose(V_ext, (0, 2, 1, 3))
    return _attention(x, wq_all, wo_all, k_hm, v_hm)


# device time: 247547 ns/iter; 1.4364x vs baseline; 1.4364x over previous
import jax
import jax.numpy as jnp
from jax import lax
from jax.experimental import pallas as pl
from jax.experimental.pallas import tpu as pltpu

N_DEV = 32
B = 2
SQ_LOC = 128
SKV = 128
H_PER = 4
DH = 64
D_MODEL = 512
CHUNK = H_PER * DH
FWD_HOPS = 16
BWD_HOPS = 15


def _fused(x, wq_shard, wo_shard, k_hm, v_hm):

    def body(x_ref, wq_ref, wo_ref, k_ref, v_ref, out_ref,
             wq_all, wo_all,
             sq_f, rq_f, so_f, ro_f,
             sq_b, rq_b, so_b, ro_b):
        me = lax.axis_index("i")
        left = (me + N_DEV - 1) % N_DEV
        right = (me + 1) % N_DEV

        x2 = x_ref[...].reshape(B * SQ_LOC, D_MODEL).astype(jnp.bfloat16)

        r = lax.broadcasted_iota(jnp.int32, (SQ_LOC, SKV), 0)
        jj = lax.broadcasted_iota(jnp.int32, (SQ_LOC, SKV), 1)
        qb = me * 2 + r // 64
        kb = jj // 64
        mask = (qb == kb) | ((qb % 4) == (kb % 4))
        row_keep = jnp.max(mask.astype(jnp.float32), axis=1, keepdims=True) > 0.5

        def compute_chunk(src):
            wq_c = wq_all[src]
            q = jnp.dot(x2, wq_c, preferred_element_type=jnp.float32)
            k4 = [k_ref[b, pl.ds(src * H_PER, H_PER), :, :] for b in range(B)]
            v4 = [v_ref[b, pl.ds(src * H_PER, H_PER), :, :] for b in range(B)]
            ctx_parts = []
            for b in range(B):
                head_parts = []
                for h in range(H_PER):
                    q_bh = q[b * SQ_LOC:(b + 1) * SQ_LOC, h * DH:(h + 1) * DH]
                    k_bh = k4[b][h]
                    v_bh = v4[b][h]
                    scores = lax.dot_general(
                        q_bh.astype(jnp.bfloat16), k_bh,
                        (((1,), (1,)), ((), ())),
                        preferred_element_type=jnp.float32,
                    ) * 0.125
                    scores = jnp.where(mask, scores, -1e9)
                    m = jnp.max(scores, axis=1, keepdims=True)
                    w = jnp.exp(scores - m)
                    s = jnp.sum(w, axis=1, keepdims=True)
                    s = jnp.where(row_keep, s, 1.0)
                    w = w / s
                    w = jnp.where(row_keep, w, 0.0)
                    head_parts.append(jnp.dot(
                        w.astype(jnp.bfloat16), v_bh,
                        preferred_element_type=jnp.float32,
                    ))
                ctx_parts.append(jnp.concatenate(head_parts, axis=1))
            ctx = jnp.concatenate(ctx_parts, axis=0).astype(jnp.bfloat16)
            contrib = jnp.dot(
                ctx, wo_all[src], preferred_element_type=jnp.float32
            )
            out_ref[...] += contrib.reshape(B, SQ_LOC, D_MODEL)

        def fwd_rdmas(h, slot):
            rq = pltpu.make_async_remote_copy(
                src_ref=wq_all.at[slot], dst_ref=wq_all.at[slot],
                send_sem=sq_f.at[h], recv_sem=rq_f.at[h],
                device_id=(right,), device_id_type=pl.DeviceIdType.MESH,
            )
            ro = pltpu.make_async_remote_copy(
                src_ref=wo_all.at[slot], dst_ref=wo_all.at[slot],
                send_sem=so_f.at[h], recv_sem=ro_f.at[h],
                device_id=(right,), device_id_type=pl.DeviceIdType.MESH,
            )
            return rq, ro

        def bwd_rdmas(h, slot):
            rq = pltpu.make_async_remote_copy(
                src_ref=wq_all.at[slot], dst_ref=wq_all.at[slot],
                send_sem=sq_b.at[h], recv_sem=rq_b.at[h],
                device_id=(left,), device_id_type=pl.DeviceIdType.MESH,
            )
            ro = pltpu.make_async_remote_copy(
                src_ref=wo_all.at[slot], dst_ref=wo_all.at[slot],
                send_sem=so_b.at[h], recv_sem=ro_b.at[h],
                device_id=(left,), device_id_type=pl.DeviceIdType.MESH,
            )
            return rq, ro

        wq_all[me] = wq_ref[...].astype(jnp.bfloat16)
        wo_all[me] = wo_ref[...].astype(jnp.bfloat16)
        out_ref[...] = jnp.zeros_like(out_ref)

        barrier = pltpu.get_barrier_semaphore()
        for nbr in (left, right):
            pl.semaphore_signal(
                barrier, inc=1, device_id=(nbr,),
                device_id_type=pl.DeviceIdType.MESH,
            )
        pl.semaphore_wait(barrier, 2)

        for h in range(FWD_HOPS):
            src_f = (me - h + N_DEV) % N_DEV
            src_b = (me + h) % N_DEV

            if h > 0:
                fq, fo = fwd_rdmas(h - 1, src_f)
                fq.wait()
                fo.wait()
                if h - 1 < BWD_HOPS:
                    bq, bo = bwd_rdmas(h - 1, src_b)
                    bq.wait()
                    bo.wait()

            fq, fo = fwd_rdmas(h, src_f)
            fq.start()
            fo.start()

            if h < BWD_HOPS:
                bq, bo = bwd_rdmas(h, src_b)
                bq.start()
                bo.start()

            compute_chunk(src_f)
            if h > 0:
                compute_chunk(src_b)

        h_last = FWD_HOPS - 1
        src_last = (me - h_last + N_DEV) % N_DEV
        fq, fo = fwd_rdmas(h_last, src_last)
        fq.wait()
        fo.wait()
        compute_chunk((me + N_DEV - FWD_HOPS) % N_DEV)

    return pl.pallas_call(
        body,
        out_shape=jax.ShapeDtypeStruct((B, SQ_LOC, D_MODEL), jnp.float32),
        in_specs=[pl.BlockSpec(memory_space=pltpu.VMEM)] * 5,
        out_specs=pl.BlockSpec(memory_space=pltpu.VMEM),
        scratch_shapes=[
            pltpu.VMEM((N_DEV, D_MODEL, CHUNK), jnp.bfloat16),
            pltpu.VMEM((N_DEV, CHUNK, D_MODEL), jnp.bfloat16),
            pltpu.SemaphoreType.DMA((FWD_HOPS,)),
            pltpu.SemaphoreType.DMA((FWD_HOPS,)),
            pltpu.SemaphoreType.DMA((FWD_HOPS,)),
            pltpu.SemaphoreType.DMA((FWD_HOPS,)),
            pltpu.SemaphoreType.DMA((BWD_HOPS,)),
            pltpu.SemaphoreType.DMA((BWD_HOPS,)),
            pltpu.SemaphoreType.DMA((BWD_HOPS,)),
            pltpu.SemaphoreType.DMA((BWD_HOPS,)),
        ],
        compiler_params=pltpu.CompilerParams(
            collective_id=0,
            vmem_limit_bytes=100 * 1024 * 1024,
        ),
    )(x, wq_shard, wo_shard, k_hm, v_hm)


def kernel(x, Wq, K_ext, V_ext, Wo):
    k_hm = jnp.transpose(K_ext, (0, 2, 1, 3)).astype(jnp.bfloat16)
    v_hm = jnp.transpose(V_ext, (0, 2, 1, 3)).astype(jnp.bfloat16)
    return _fused(x, Wq, Wo, k_hm, v_hm)
